# baseline (speedup 1.0000x reference)
import jax
import jax.numpy as jnp
from jax import lax
from jax.experimental import pallas as pl
from jax.experimental.pallas import tpu as pltpu
from jax.experimental.pallas import tpu_sc as plsc

_ROWS = 16384
_COLS = 256
_HALF = _COLS // 2

_INFO = plsc.get_sparse_core_info()
_NC = _INFO.num_cores
_NS = _INFO.num_subcores
_NW = _NC * _NS
_L = _INFO.num_lanes

_ROWS_PER_W = _ROWS // _NW
_CHUNK_ROWS = 64
_NCHUNK = _ROWS_PER_W // _CHUNK_ROWS
_QPR = _COLS // (2 * _L)


def _body(in_hbm, even_hbm, odd_hbm,
          in0, in1, e0, e1, o0, o1,
          sin0, sin1, se0, se1, so0, so1):
    wid = lax.axis_index("s") * _NC + lax.axis_index("c")
    lane = lax.iota(jnp.int32, _L)
    cols = [lane * 2 + q * (2 * _L) for q in range(_QPR)]

    ins = (in0, in1)
    ebufs = (e0, e1)
    obufs = (o0, o1)
    sins = (sin0, sin1)
    ses = (se0, se1)
    sos = (so0, so1)

    def row0(c):
        return wid * _ROWS_PER_W + c * _CHUNK_ROWS

    def start_in(c):
        return pltpu.async_copy(
            in_hbm.at[pl.ds(row0(c), _CHUNK_ROWS), :], ins[c % 2],
            sins[c % 2])

    in_copies = [start_in(0)]
    out_copies = [None, None]
    for c in range(_NCHUNK):
        b = c % 2
        if c + 1 < _NCHUNK:
            in_copies.append(start_in(c + 1))
        in_copies[c].wait()
        if out_copies[b] is not None:
            for cp in out_copies[b]:
                cp.wait()
        in_buf, ebuf, obuf = ins[b], ebufs[b], obufs[b]

        @plsc.parallel_loop(0, _CHUNK_ROWS, 1, unroll=2)
        def _(r):
            rvec = jnp.broadcast_to(r, (_L,))
            for q in range(_QPR):
                ev = plsc.load_gather(in_buf, [rvec, cols[q]])
                od = plsc.load_gather(in_buf, [rvec, cols[q] + 1])
                ebuf[r, pl.ds(q * _L, _L)] = ev
                obuf[r, pl.ds(q * _L, _L)] = od

        out_copies[b] = (
            pltpu.async_copy(
                ebuf, even_hbm.at[pl.ds(row0(c), _CHUNK_ROWS), :], ses[b]),
            pltpu.async_copy(
                obuf, odd_hbm.at[pl.ds(row0(c), _CHUNK_ROWS), :], sos[b]),
        )
    for cps in out_copies:
        for cp in cps:
            cp.wait()


@jax.jit
def _split(x):
    mesh = plsc.VectorSubcoreMesh(core_axis_name="c", subcore_axis_name="s")
    f = pl.kernel(
        _body,
        out_type=[
            jax.ShapeDtypeStruct((_ROWS, _HALF), jnp.float32),
            jax.ShapeDtypeStruct((_ROWS, _HALF), jnp.float32),
        ],
        mesh=mesh,
        scratch_types=[
            pltpu.VMEM((_CHUNK_ROWS, _COLS), jnp.float32),
            pltpu.VMEM((_CHUNK_ROWS, _COLS), jnp.float32),
            pltpu.VMEM((_CHUNK_ROWS, _HALF), jnp.float32),
            pltpu.VMEM((_CHUNK_ROWS, _HALF), jnp.float32),
            pltpu.VMEM((_CHUNK_ROWS, _HALF), jnp.float32),
            pltpu.VMEM((_CHUNK_ROWS, _HALF), jnp.float32),
            pltpu.SemaphoreType.DMA,
            pltpu.SemaphoreType.DMA,
            pltpu.SemaphoreType.DMA,
            pltpu.SemaphoreType.DMA,
            pltpu.SemaphoreType.DMA,
            pltpu.SemaphoreType.DMA,
        ],
        compiler_params=pltpu.CompilerParams(needs_layout_passes=False),
    )
    return f(x)


def kernel(inputs, shape_indices, energy_indices):
    del shape_indices, energy_indices
    even, odd = _split(inputs)
    return (even, odd)
